# baseline (speedup 1.0000x reference)
import functools

import jax
import jax.numpy as jnp
from jax import lax
from jax.experimental import pallas as pl
from jax.experimental.pallas import tpu as pltpu
from jax.experimental.pallas import tpu_sc as plsc

N = 100000
E = 1600000
HID = 32
NC = 2
NS = 16
SUB = 128
NSUB = 8
CHUNK = SUB * NSUB
NPAD = 102400
EPAD = 1605632
EROWS = EPAD // SUB
ROWS_PER_TILE = NPAD // NS
BLK = 2048
NBLK = NPAD // BLK


def _make_edge_pass(split_edges_by_core: bool, table_rows: int):
    if split_edges_by_core:
        per_worker = EPAD // (NC * NS)
    else:
        per_worker = EPAD // NS
    n_chunks = per_worker // CHUNK
    rows_per_worker = per_worker // SUB

    mesh = plsc.VectorSubcoreMesh(core_axis_name="c", subcore_axis_name="s")

    @functools.partial(
        pl.kernel,
        out_type=jax.ShapeDtypeStruct((NC, NPAD, 16), jnp.float32),
        mesh=mesh,
        compiler_params=pltpu.CompilerParams(use_tc_tiling_on_sc=False),
        scratch_types=[
            pltpu.VMEM((CHUNK,), jnp.int32),
            pltpu.VMEM((CHUNK,), jnp.int32),
            pltpu.VMEM((CHUNK, 16), jnp.float32),
            pltpu.VMEM_SHARED((NPAD, 16), jnp.float32),
            pltpu.SemaphoreType.DMA,
            pltpu.SemaphoreType.DMA,
        ],
    )
    def kern(table_hbm, srcs_hbm, dst_hbm, zeros_hbm, out_hbm,
             srcb, dstb, rows, acc, gsem, ssem):
        c = lax.axis_index("c")
        s = lax.axis_index("s")
        r0 = s * ROWS_PER_TILE
        pltpu.sync_copy(zeros_hbm.at[pl.ds(r0, ROWS_PER_TILE)],
                        acc.at[pl.ds(r0, ROWS_PER_TILE)])
        plsc.subcore_barrier()

        if split_edges_by_core:
            wid = s * NC + c
        else:
            wid = s
        base_row = wid * rows_per_worker

        def chunk_body(i, carry):
            e0 = (base_row + i * NSUB) * SUB
            pltpu.sync_copy(srcs_hbm.at[c, pl.ds(e0, CHUNK)], srcb)
            pltpu.sync_copy(dst_hbm.at[pl.ds(e0, CHUNK)], dstb)
            pltpu.async_copy(table_hbm.at[srcb], rows, gsem).wait()
            pltpu.async_copy(rows, acc.at[dstb], ssem, add=True).wait()
            return carry

        lax.fori_loop(0, n_chunks, chunk_body, 0)
        plsc.subcore_barrier()
        pltpu.sync_copy(acc.at[pl.ds(r0, ROWS_PER_TILE)],
                        out_hbm.at[c, pl.ds(r0, ROWS_PER_TILE)])

    return kern


_edge_pass0 = _make_edge_pass(True, NPAD)
_edge_pass1 = _make_edge_pass(False, 2 * NPAD)


def _dense1_body(p0_ref, ages_ref, wl0_ref, wr0_ref, b0_ref,
                 table_ref, deg_ref):
    ssum = p0_ref[0] + p0_ref[1]
    agg0 = ssum[:, 0:1]
    deg = ssum[:, 1:2]
    a = agg0 / jnp.maximum(deg, 1.0)
    h1 = a * wl0_ref[...] + ages_ref[...] * wr0_ref[...] + b0_ref[...]
    table_ref[...] = jnp.maximum(h1, 0.0)
    deg_ref[...] = deg


def _dense1(p0, ages_pad, wl0, wr0, b0):
    return pl.pallas_call(
        _dense1_body,
        grid=(NBLK,),
        in_specs=[
            pl.BlockSpec((NC, BLK, 16), lambda i: (0, i, 0)),
            pl.BlockSpec((BLK, 1), lambda i: (i, 0)),
            pl.BlockSpec((1, HID), lambda i: (0, 0)),
            pl.BlockSpec((1, HID), lambda i: (0, 0)),
            pl.BlockSpec((1, HID), lambda i: (0, 0)),
        ],
        out_specs=[
            pl.BlockSpec((BLK, HID), lambda i: (i, 0)),
            pl.BlockSpec((BLK, 1), lambda i: (i, 0)),
        ],
        out_shape=[
            jax.ShapeDtypeStruct((NPAD, HID), jnp.float32),
            jax.ShapeDtypeStruct((NPAD, 1), jnp.float32),
        ],
    )(p0, ages_pad, wl0, wr0, b0)


def _hdot(a, b):
    return jnp.dot(a, b, preferred_element_type=jnp.float32,
                   precision=lax.Precision.HIGHEST)


def _dense2_body(p1_ref, table_ref, deg_ref, wl1a_ref, wl1b_ref, wr1_ref,
                 b1_ref, wv_ref, bv_ref, tt_ref, wm1a_ref, wm1b_ref,
                 bm1_ref, wm2_ref, bm2_ref, acc_ref, res_ref):
    i = pl.program_id(0)
    inv_deg = 1.0 / jnp.maximum(deg_ref[...], 1.0)
    a1a = p1_ref[0] * inv_deg
    a1b = p1_ref[1] * inv_deg
    h1 = table_ref[...]
    h2 = (_hdot(a1a, wl1a_ref[...]) + _hdot(a1b, wl1b_ref[...])
          + _hdot(h1, wr1_ref[...]) + b1_ref[...])
    h2 = jnp.maximum(h2, 0.0)
    rows = lax.broadcasted_iota(jnp.int32, (BLK, 1), 0) + i * BLK
    h2 = jnp.where(rows < N, h2, 0.0)
    part = jnp.sum(h2, axis=0, keepdims=True)

    @pl.when(i == 0)
    def _init():
        acc_ref[...] = jnp.zeros_like(acc_ref)

    acc_ref[...] += part

    @pl.when(i == NBLK - 1)
    def _final():
        cpool = acc_ref[...] * (1.0 / N)
        z = _hdot(cpool, wv_ref[...]) + bv_ref[...]
        hm = (_hdot(z, wm1a_ref[...]) + _hdot(tt_ref[...], wm1b_ref[...])
              + bm1_ref[...])
        hm = jnp.maximum(hm, 0.0)
        o = _hdot(hm, wm2_ref[...]) + bm2_ref[...]
        t = jnp.exp(-jnp.abs(o))
        l1p = jnp.where(t < 1e-3, t * (1.0 - 0.5 * t), jnp.log(1.0 + t))
        res_ref[...] = jnp.maximum(o, 0.0) + l1p

    return


def _dense2(p1, table, degv, wl1a, wl1b, wr1, b1, wv, bv, tt,
            wm1a, wm1b, bm1, wm2, bm2):
    full = lambda shape: pl.BlockSpec(shape, lambda i: tuple(0 for _ in shape))
    _, res = pl.pallas_call(
        _dense2_body,
        grid=(NBLK,),
        in_specs=[
            pl.BlockSpec((NC, BLK, 16), lambda i: (0, i, 0)),
            pl.BlockSpec((BLK, HID), lambda i: (i, 0)),
            pl.BlockSpec((BLK, 1), lambda i: (i, 0)),
            full((16, HID)), full((16, HID)), full((HID, HID)),
            full((1, HID)), full((HID, 5)), full((1, 5)), full((1, 2)),
            full((5, HID)), full((2, HID)), full((1, HID)),
            full((HID, 1)), full((1, 1)),
        ],
        out_specs=[
            pl.BlockSpec((1, HID), lambda i: (0, 0)),
            pl.BlockSpec((1, 1), lambda i: (0, 0)),
        ],
        out_shape=[
            jax.ShapeDtypeStruct((1, HID), jnp.float32),
            jax.ShapeDtypeStruct((1, 1), jnp.float32),
        ],
    )(p1, table, degv, wl1a, wl1b, wr1, b1, wv, bv, tt,
      wm1a, wm1b, bm1, wm2, bm2)
    return res


def kernel(ages, edge_index, batch, temp, t, Wl0, Wr0, b0, Wl1, Wr1, b1,
           Wv, bv, Wm1, bm1, Wm2, bm2):
    del batch
    ages = ages.astype(jnp.float32)
    src = edge_index[0]
    dst = edge_index[1]
    padlen = EPAD - E
    src_p = jnp.concatenate([src, jnp.zeros((padlen,), jnp.int32)])
    dst_p = jnp.concatenate([dst, jnp.full((padlen,), N, jnp.int32)])
    srcs0 = jnp.stack([src_p, src_p])
    srcs1 = jnp.stack([src_p, src_p + NPAD])
    dst2d = dst_p
    zeros_n = jnp.zeros((NPAD, 16), jnp.float32)

    ages_pad = jnp.pad(ages.reshape(N, 1), ((0, NPAD - N), (0, 0)))
    table0 = jnp.concatenate(
        [ages_pad, jnp.ones((NPAD, 1), jnp.float32),
         jnp.zeros((NPAD, 14), jnp.float32)], axis=1)

    p0 = _edge_pass0(table0, srcs0, dst2d, zeros_n)
    table, degv = _dense1(p0, ages_pad, Wl0, Wr0, b0.reshape(1, HID))

    table_sc = table.reshape(NPAD, 2, 16).transpose(1, 0, 2).reshape(
        2 * NPAD, 16)
    p1 = _edge_pass1(table_sc, srcs1, dst2d, zeros_n)

    tt = jnp.stack([jnp.asarray(temp), jnp.asarray(t)]).astype(
        jnp.float32).reshape(1, 2)
    res = _dense2(p1, table, degv, Wl1[:16], Wl1[16:], Wr1,
                  b1.reshape(1, HID), Wv, bv.reshape(1, 5), tt,
                  Wm1[:5], Wm1[5:], bm1.reshape(1, HID), Wm2,
                  bm2.reshape(1, 1))
    return res.reshape(-1)


# speedup vs baseline: 2.1227x; 2.1227x over previous
import functools

import jax
import jax.numpy as jnp
from jax import lax
from jax.experimental import pallas as pl
from jax.experimental.pallas import tpu as pltpu
from jax.experimental.pallas import tpu_sc as plsc

N = 100000
E = 1600000
HID = 32
NC = 2
NS = 16
SUB = 128
NSUB = 8
EROWS = E // SUB
NPAD = 102400
ROWS_PER_TILE = NPAD // NS
FR = NPAD // 8
BR = 1600
NBLK = FR // BR


def _make_edge_pass(split_edges_by_core: bool, table_rows: int,
                    core_stride: int):
    if split_edges_by_core:
        nworkers = NC * NS
    else:
        nworkers = NS
    rows_even = EROWS // nworkers
    rows_last = EROWS - rows_even * (nworkers - 1)

    mesh = plsc.VectorSubcoreMesh(core_axis_name="c", subcore_axis_name="s")

    @functools.partial(
        pl.kernel,
        out_type=jax.ShapeDtypeStruct((NC, NPAD, 16), jnp.float32),
        mesh=mesh,
        compiler_params=pltpu.CompilerParams(use_tc_tiling_on_sc=False),
        scratch_types=[
            pltpu.VMEM((NSUB, SUB), jnp.int32),
            pltpu.VMEM((NSUB, SUB), jnp.int32),
            pltpu.VMEM((NSUB, SUB, 16), jnp.float32),
            pltpu.VMEM_SHARED((NPAD, 16), jnp.float32),
            pltpu.SemaphoreType.DMA((NSUB,)),
            pltpu.SemaphoreType.DMA((NSUB,)),
        ],
    )
    def kern(table_hbm, ei_hbm, zeros_hbm, out_hbm,
             srcb, dstb, rows, acc, gsem, ssem):
        c = lax.axis_index("c")
        s = lax.axis_index("s")
        r0 = s * ROWS_PER_TILE
        pltpu.sync_copy(zeros_hbm.at[pl.ds(r0, ROWS_PER_TILE)],
                        acc.at[pl.ds(r0, ROWS_PER_TILE)])
        plsc.subcore_barrier()

        if split_edges_by_core:
            wid = s * NC + c
        else:
            wid = s
        base_row = wid * rows_even
        my_rows = jnp.where(wid == nworkers - 1, rows_last, rows_even)
        n_full = my_rows // NSUB
        n_tail = my_rows - n_full * NSUB
        off = c * core_stride

        def add_offset(nrows):
            for j in range(nrows):
                for k in range(SUB // 16):
                    sl = pl.ds(k * 16, 16)
                    srcb[j, sl] = srcb[j, sl] + off

        def chunk_body(i, carry):
            cr = base_row + i * NSUB
            pltpu.sync_copy(ei_hbm.at[0, pl.ds(cr, NSUB)], srcb)
            pltpu.sync_copy(ei_hbm.at[1, pl.ds(cr, NSUB)], dstb)
            if core_stride:
                add_offset(NSUB)
            gd = [pltpu.async_copy(table_hbm.at[srcb.at[j]], rows.at[j],
                                   gsem.at[j])
                  for j in range(NSUB)]
            sd = []
            for j in range(NSUB):
                gd[j].wait()
                sd.append(pltpu.async_copy(rows.at[j], acc.at[dstb.at[j]],
                                           ssem.at[j], add=True))
            for d in sd:
                d.wait()
            return carry

        lax.fori_loop(0, n_full, chunk_body, 0)

        def tail_body(i, carry):
            cr = base_row + n_full * NSUB + i
            pltpu.sync_copy(ei_hbm.at[0, pl.ds(cr, 1)],
                            srcb.at[pl.ds(0, 1)])
            pltpu.sync_copy(ei_hbm.at[1, pl.ds(cr, 1)],
                            dstb.at[pl.ds(0, 1)])
            if core_stride:
                add_offset(1)
            pltpu.async_copy(table_hbm.at[srcb.at[0]], rows.at[0],
                             gsem.at[0]).wait()
            pltpu.async_copy(rows.at[0], acc.at[dstb.at[0]], ssem.at[0],
                             add=True).wait()
            return carry

        lax.fori_loop(0, n_tail, tail_body, 0)

        plsc.subcore_barrier()
        pltpu.sync_copy(acc.at[pl.ds(r0, ROWS_PER_TILE)],
                        out_hbm.at[c, pl.ds(r0, ROWS_PER_TILE)])

    return kern


_edge_pass0 = _make_edge_pass(True, NPAD, 0)
_edge_pass1 = _make_edge_pass(False, 2 * NPAD, NPAD)


def _hdot(a, b):
    return jnp.dot(a, b, preferred_element_type=jnp.float32,
                   precision=lax.Precision.HIGHEST)


def _dense1_body(p0_ref, t0_ref, s0_ref, s1_ref, wl0a_ref, wl0b_ref,
                 wr0a_ref, wr0b_ref, b0a_ref, b0b_ref, table_ref, deg_ref):
    ssum = p0_ref[0] + p0_ref[1]
    deg_f = _hdot(ssum, s1_ref[...])
    agg_f = _hdot(ssum, s0_ref[...])
    x_f = _hdot(t0_ref[...], s0_ref[...])
    a_f = agg_f / jnp.maximum(deg_f, 1.0)
    h1a = jnp.maximum(a_f * wl0a_ref[...] + x_f * wr0a_ref[...]
                      + b0a_ref[...], 0.0)
    h1b = jnp.maximum(a_f * wl0b_ref[...] + x_f * wr0b_ref[...]
                      + b0b_ref[...], 0.0)
    table_ref[...] = jnp.stack([h1a, h1b])
    deg_ref[...] = deg_f


def _dense1(p0f, table0f, s0, s1, wl0a, wl0b, wr0a, wr0b, b0a, b0b):
    rowf = pl.BlockSpec((BR, 128), lambda i: (i, 0))
    full = lambda shape: pl.BlockSpec(shape, lambda i: tuple(0 for _ in shape))
    return pl.pallas_call(
        _dense1_body,
        grid=(NBLK,),
        in_specs=[
            pl.BlockSpec((NC, BR, 128), lambda i: (0, i, 0)),
            rowf, full((128, 128)), full((128, 128)),
            full((1, 128)), full((1, 128)), full((1, 128)), full((1, 128)),
            full((1, 128)), full((1, 128)),
        ],
        out_specs=[
            pl.BlockSpec((NC, BR, 128), lambda i: (0, i, 0)),
            rowf,
        ],
        out_shape=[
            jax.ShapeDtypeStruct((NC, FR, 128), jnp.float32),
            jax.ShapeDtypeStruct((FR, 128), jnp.float32),
        ],
    )(p0f, table0f, s0, s1, wl0a, wl0b, wr0a, wr0b, b0a, b0b)


def _dense2_body(p1_ref, table_ref, deg_ref, bdA_ref, bdB_ref, b1a_ref,
                 b1b_ref, fold_ref, wva_ref, wvb_ref, bv_ref, tt_ref,
                 wm1a_ref, wm1b_ref, bm1_ref, wm2_ref, bm2_ref,
                 accA_ref, accB_ref, res_ref):
    i = pl.program_id(0)
    inv_deg = 1.0 / jnp.maximum(deg_ref[...], 1.0)
    a1a = p1_ref[0] * inv_deg
    a1b = p1_ref[1] * inv_deg
    h2a = jnp.maximum(
        _hdot(a1a, bdA_ref[0]) + _hdot(a1b, bdA_ref[1])
        + _hdot(table_ref[0], bdA_ref[2]) + _hdot(table_ref[1], bdA_ref[3])
        + b1a_ref[...], 0.0)
    h2b = jnp.maximum(
        _hdot(a1a, bdB_ref[0]) + _hdot(a1b, bdB_ref[1])
        + _hdot(table_ref[0], bdB_ref[2]) + _hdot(table_ref[1], bdB_ref[3])
        + b1b_ref[...], 0.0)
    rows = lax.broadcasted_iota(jnp.int32, (BR, 128), 0) + i * BR
    mask = rows < (N // 8)
    h2a = jnp.where(mask, h2a, 0.0)
    h2b = jnp.where(mask, h2b, 0.0)

    @pl.when(i == 0)
    def _init():
        accA_ref[...] = jnp.zeros_like(accA_ref)
        accB_ref[...] = jnp.zeros_like(accB_ref)

    accA_ref[...] += jnp.sum(h2a, axis=0, keepdims=True)
    accB_ref[...] += jnp.sum(h2b, axis=0, keepdims=True)

    @pl.when(i == NBLK - 1)
    def _final():
        cA = _hdot(accA_ref[...], fold_ref[...]) * (1.0 / N)
        cB = _hdot(accB_ref[...], fold_ref[...]) * (1.0 / N)
        z = _hdot(cA, wva_ref[...]) + _hdot(cB, wvb_ref[...]) + bv_ref[...]
        hm = (_hdot(z, wm1a_ref[...]) + _hdot(tt_ref[...], wm1b_ref[...])
              + bm1_ref[...])
        hm = jnp.maximum(hm, 0.0)
        o = _hdot(hm, wm2_ref[...]) + bm2_ref[...]
        t = jnp.exp(-jnp.abs(o))
        l1p = jnp.where(t < 1e-3, t * (1.0 - 0.5 * t), jnp.log(1.0 + t))
        res_ref[...] = jnp.maximum(o, 0.0) + l1p


def _dense2(p1f, tablef, degf, bdA, bdB, b1a, b1b, fold, wva, wvb, bv, tt,
            wm1a, wm1b, bm1, wm2, bm2):
    full = lambda shape: pl.BlockSpec(shape, lambda i: tuple(0 for _ in shape))
    _, _, res = pl.pallas_call(
        _dense2_body,
        grid=(NBLK,),
        in_specs=[
            pl.BlockSpec((NC, BR, 128), lambda i: (0, i, 0)),
            pl.BlockSpec((NC, BR, 128), lambda i: (0, i, 0)),
            pl.BlockSpec((BR, 128), lambda i: (i, 0)),
            full((4, 128, 128)), full((4, 128, 128)),
            full((1, 128)), full((1, 128)), full((128, 16)),
            full((16, 5)), full((16, 5)), full((1, 5)),
            full((1, 2)), full((5, HID)), full((2, HID)), full((1, HID)),
            full((HID, 1)), full((1, 1)),
        ],
        out_specs=[
            pl.BlockSpec((1, 128), lambda i: (0, 0)),
            pl.BlockSpec((1, 128), lambda i: (0, 0)),
            pl.BlockSpec((1, 1), lambda i: (0, 0)),
        ],
        out_shape=[
            jax.ShapeDtypeStruct((1, 128), jnp.float32),
            jax.ShapeDtypeStruct((1, 128), jnp.float32),
            jax.ShapeDtypeStruct((1, 1), jnp.float32),
        ],
    )(p1f, tablef, degf, bdA, bdB, b1a, b1b, fold, wva, wvb, bv, tt,
      wm1a, wm1b, bm1, wm2, bm2)
    return res


def kernel(ages, edge_index, batch, temp, t, Wl0, Wr0, b0, Wl1, Wr1, b1,
           Wv, bv, Wm1, bm1, Wm2, bm2):
    del batch
    ages = ages.astype(jnp.float32)
    ei = edge_index.reshape(2, EROWS, SUB)
    zeros_n = jnp.zeros((NPAD, 16), jnp.float32)

    ages_pad = jnp.pad(ages.reshape(N, 1), ((0, NPAD - N), (0, 0)))
    table0 = jnp.concatenate(
        [ages_pad, jnp.ones((NPAD, 1), jnp.float32),
         jnp.zeros((NPAD, 14), jnp.float32)], axis=1)
    eye8 = jnp.eye(8, dtype=jnp.float32)
    sel = lambda j: jnp.kron(
        eye8, jnp.zeros((16, 16), jnp.float32).at[j].set(1.0))
    tile8 = lambda v: jnp.tile(v.reshape(1, 16), (1, 8))

    p0 = _edge_pass0(table0, ei, zeros_n)
    wl0 = Wl0.reshape(HID)
    wr0 = Wr0.reshape(HID)
    table, degf = _dense1(
        p0.reshape(NC, FR, 128), table0.reshape(FR, 128), sel(0), sel(1),
        tile8(wl0[:16]), tile8(wl0[16:]), tile8(wr0[:16]), tile8(wr0[16:]),
        tile8(b0[:16]), tile8(b0[16:]))

    p1 = _edge_pass1(table.reshape(2 * NPAD, 16), ei, zeros_n)

    bd = lambda w: jnp.kron(eye8, w)
    bdA = jnp.stack([bd(Wl1[:16, :16]), bd(Wl1[16:, :16]),
                     bd(Wr1[:16, :16]), bd(Wr1[16:, :16])])
    bdB = jnp.stack([bd(Wl1[:16, 16:]), bd(Wl1[16:, 16:]),
                     bd(Wr1[:16, 16:]), bd(Wr1[16:, 16:])])
    fold = jnp.kron(jnp.ones((8, 1), jnp.float32),
                    jnp.eye(16, dtype=jnp.float32))
    tt = jnp.stack([jnp.asarray(temp), jnp.asarray(t)]).astype(
        jnp.float32).reshape(1, 2)
    res = _dense2(p1.reshape(NC, FR, 128), table, degf,
                  bdA, bdB, tile8(b1[:16]), tile8(b1[16:]), fold,
                  Wv[:16], Wv[16:], bv.reshape(1, 5), tt,
                  Wm1[:5], Wm1[5:], bm1.reshape(1, HID), Wm2,
                  bm2.reshape(1, 1))
    return res.reshape(-1)


# speedup vs baseline: 2.3937x; 1.1277x over previous
import functools

import jax
import jax.numpy as jnp
from jax import lax
from jax.experimental import pallas as pl
from jax.experimental.pallas import tpu as pltpu
from jax.experimental.pallas import tpu_sc as plsc

N = 100000
E = 1600000
HID = 32
NC = 2
NS = 16
SUB = 128
NSUB = 12
EROWS = E // SUB
NPAD = 102400
ROWS_PER_TILE = NPAD // NS
FR = NPAD // 8
BR = 1600
NBLK = FR // BR


def _make_edge_pass(split_edges_by_core: bool, table_rows: int,
                    core_stride: int):
    if split_edges_by_core:
        nworkers = NC * NS
    else:
        nworkers = NS
    rows_even = EROWS // nworkers
    rows_last = EROWS - rows_even * (nworkers - 1)

    mesh = plsc.VectorSubcoreMesh(core_axis_name="c", subcore_axis_name="s")

    @functools.partial(
        pl.kernel,
        out_type=jax.ShapeDtypeStruct((NC, NPAD, 16), jnp.float32),
        mesh=mesh,
        compiler_params=pltpu.CompilerParams(use_tc_tiling_on_sc=False),
        scratch_types=[
            pltpu.VMEM((NSUB, SUB), jnp.int32),
            pltpu.VMEM((NSUB, SUB), jnp.int32),
            pltpu.VMEM((NSUB, SUB, 16), jnp.float32),
            pltpu.VMEM_SHARED((NPAD, 16), jnp.float32),
            pltpu.SemaphoreType.DMA((NSUB,)),
            pltpu.SemaphoreType.DMA((NSUB,)),
        ],
    )
    def kern(table_hbm, ei_hbm, zeros_hbm, out_hbm,
             srcb, dstb, rows, acc, gsem, ssem):
        c = lax.axis_index("c")
        s = lax.axis_index("s")
        r0 = s * ROWS_PER_TILE
        pltpu.sync_copy(zeros_hbm.at[pl.ds(r0, ROWS_PER_TILE)],
                        acc.at[pl.ds(r0, ROWS_PER_TILE)])
        plsc.subcore_barrier()

        if split_edges_by_core:
            wid = s * NC + c
        else:
            wid = s
        base_row = wid * rows_even
        my_rows = jnp.where(wid == nworkers - 1, rows_last, rows_even)
        n_full = my_rows // NSUB
        n_tail = my_rows - n_full * NSUB
        off = c * core_stride

        def add_offset(nrows):
            for j in range(nrows):
                for k in range(SUB // 16):
                    sl = pl.ds(k * 16, 16)
                    srcb[j, sl] = srcb[j, sl] + off

        def chunk_body(i, carry):
            cr = base_row + i * NSUB
            pltpu.sync_copy(ei_hbm.at[0, pl.ds(cr, NSUB)], srcb)
            pltpu.sync_copy(ei_hbm.at[1, pl.ds(cr, NSUB)], dstb)
            if core_stride:
                add_offset(NSUB)
            gd = [pltpu.async_copy(table_hbm.at[srcb.at[j]], rows.at[j],
                                   gsem.at[j])
                  for j in range(NSUB)]
            sd = []
            for j in range(NSUB):
                gd[j].wait()
                sd.append(pltpu.async_copy(rows.at[j], acc.at[dstb.at[j]],
                                           ssem.at[j], add=True))
            for d in sd:
                d.wait()
            return carry

        lax.fori_loop(0, n_full, chunk_body, 0)

        def tail_body(i, carry):
            cr = base_row + n_full * NSUB + i
            pltpu.sync_copy(ei_hbm.at[0, pl.ds(cr, 1)],
                            srcb.at[pl.ds(0, 1)])
            pltpu.sync_copy(ei_hbm.at[1, pl.ds(cr, 1)],
                            dstb.at[pl.ds(0, 1)])
            if core_stride:
                add_offset(1)
            pltpu.async_copy(table_hbm.at[srcb.at[0]], rows.at[0],
                             gsem.at[0]).wait()
            pltpu.async_copy(rows.at[0], acc.at[dstb.at[0]], ssem.at[0],
                             add=True).wait()
            return carry

        lax.fori_loop(0, n_tail, tail_body, 0)

        plsc.subcore_barrier()
        pltpu.sync_copy(acc.at[pl.ds(r0, ROWS_PER_TILE)],
                        out_hbm.at[c, pl.ds(r0, ROWS_PER_TILE)])

    return kern


_edge_pass0 = _make_edge_pass(True, NPAD, 0)
_edge_pass1 = _make_edge_pass(False, 2 * NPAD, NPAD)


def _hdot(a, b):
    return jnp.dot(a, b, preferred_element_type=jnp.float32,
                   precision=lax.Precision.HIGHEST)


_bdot = _hdot


def _dense1_body(p0_ref, t0_ref, s0_ref, s1_ref, wl0a_ref, wl0b_ref,
                 wr0a_ref, wr0b_ref, b0a_ref, b0b_ref, table_ref, deg_ref):
    ssum = p0_ref[0] + p0_ref[1]
    deg_f = _bdot(ssum, s1_ref[...])
    agg_f = _bdot(ssum, s0_ref[...])
    x_f = _bdot(t0_ref[...], s0_ref[...])
    a_f = agg_f / jnp.maximum(deg_f, 1.0)
    h1a = jnp.maximum(a_f * wl0a_ref[...] + x_f * wr0a_ref[...]
                      + b0a_ref[...], 0.0)
    h1b = jnp.maximum(a_f * wl0b_ref[...] + x_f * wr0b_ref[...]
                      + b0b_ref[...], 0.0)
    table_ref[...] = jnp.stack([h1a, h1b])
    deg_ref[...] = deg_f


def _dense1(p0f, table0f, s0, s1, wl0a, wl0b, wr0a, wr0b, b0a, b0b):
    rowf = pl.BlockSpec((BR, 128), lambda i: (i, 0))
    full = lambda shape: pl.BlockSpec(shape, lambda i: tuple(0 for _ in shape))
    return pl.pallas_call(
        _dense1_body,
        grid=(NBLK,),
        in_specs=[
            pl.BlockSpec((NC, BR, 128), lambda i: (0, i, 0)),
            rowf, full((128, 128)), full((128, 128)),
            full((1, 128)), full((1, 128)), full((1, 128)), full((1, 128)),
            full((1, 128)), full((1, 128)),
        ],
        out_specs=[
            pl.BlockSpec((NC, BR, 128), lambda i: (0, i, 0)),
            rowf,
        ],
        out_shape=[
            jax.ShapeDtypeStruct((NC, FR, 128), jnp.float32),
            jax.ShapeDtypeStruct((FR, 128), jnp.float32),
        ],
    )(p0f, table0f, s0, s1, wl0a, wl0b, wr0a, wr0b, b0a, b0b)


def _dense2_body(p1_ref, table_ref, deg_ref, bdA_ref, bdB_ref, b1a_ref,
                 b1b_ref, fold_ref, wva_ref, wvb_ref, bv_ref, tt_ref,
                 wm1a_ref, wm1b_ref, bm1_ref, wm2_ref, bm2_ref,
                 accA_ref, accB_ref, res_ref):
    i = pl.program_id(0)
    inv_deg = 1.0 / jnp.maximum(deg_ref[...], 1.0)
    a1a = p1_ref[0] * inv_deg
    a1b = p1_ref[1] * inv_deg
    h2a = jnp.maximum(
        _bdot(a1a, bdA_ref[0]) + _bdot(a1b, bdA_ref[1])
        + _bdot(table_ref[0], bdA_ref[2]) + _bdot(table_ref[1], bdA_ref[3])
        + b1a_ref[...], 0.0)
    h2b = jnp.maximum(
        _bdot(a1a, bdB_ref[0]) + _bdot(a1b, bdB_ref[1])
        + _bdot(table_ref[0], bdB_ref[2]) + _bdot(table_ref[1], bdB_ref[3])
        + b1b_ref[...], 0.0)
    rows = lax.broadcasted_iota(jnp.int32, (BR, 128), 0) + i * BR
    mask = rows < (N // 8)
    h2a = jnp.where(mask, h2a, 0.0)
    h2b = jnp.where(mask, h2b, 0.0)

    @pl.when(i == 0)
    def _init():
        accA_ref[...] = jnp.zeros_like(accA_ref)
        accB_ref[...] = jnp.zeros_like(accB_ref)

    accA_ref[...] += jnp.sum(h2a, axis=0, keepdims=True)
    accB_ref[...] += jnp.sum(h2b, axis=0, keepdims=True)

    @pl.when(i == NBLK - 1)
    def _final():
        cA = _hdot(accA_ref[...], fold_ref[...]) * (1.0 / N)
        cB = _hdot(accB_ref[...], fold_ref[...]) * (1.0 / N)
        z = _hdot(cA, wva_ref[...]) + _hdot(cB, wvb_ref[...]) + bv_ref[...]
        hm = (_hdot(z, wm1a_ref[...]) + _hdot(tt_ref[...], wm1b_ref[...])
              + bm1_ref[...])
        hm = jnp.maximum(hm, 0.0)
        o = _hdot(hm, wm2_ref[...]) + bm2_ref[...]
        t = jnp.exp(-jnp.abs(o))
        l1p = jnp.where(t < 1e-3, t * (1.0 - 0.5 * t), jnp.log(1.0 + t))
        res_ref[...] = jnp.maximum(o, 0.0) + l1p


def _dense2(p1f, tablef, degf, bdA, bdB, b1a, b1b, fold, wva, wvb, bv, tt,
            wm1a, wm1b, bm1, wm2, bm2):
    full = lambda shape: pl.BlockSpec(shape, lambda i: tuple(0 for _ in shape))
    _, _, res = pl.pallas_call(
        _dense2_body,
        grid=(NBLK,),
        in_specs=[
            pl.BlockSpec((NC, BR, 128), lambda i: (0, i, 0)),
            pl.BlockSpec((NC, BR, 128), lambda i: (0, i, 0)),
            pl.BlockSpec((BR, 128), lambda i: (i, 0)),
            full((4, 128, 128)), full((4, 128, 128)),
            full((1, 128)), full((1, 128)), full((128, 16)),
            full((16, 5)), full((16, 5)), full((1, 5)),
            full((1, 2)), full((5, HID)), full((2, HID)), full((1, HID)),
            full((HID, 1)), full((1, 1)),
        ],
        out_specs=[
            pl.BlockSpec((1, 128), lambda i: (0, 0)),
            pl.BlockSpec((1, 128), lambda i: (0, 0)),
            pl.BlockSpec((1, 1), lambda i: (0, 0)),
        ],
        out_shape=[
            jax.ShapeDtypeStruct((1, 128), jnp.float32),
            jax.ShapeDtypeStruct((1, 128), jnp.float32),
            jax.ShapeDtypeStruct((1, 1), jnp.float32),
        ],
    )(p1f, tablef, degf, bdA, bdB, b1a, b1b, fold, wva, wvb, bv, tt,
      wm1a, wm1b, bm1, wm2, bm2)
    return res


def kernel(ages, edge_index, batch, temp, t, Wl0, Wr0, b0, Wl1, Wr1, b1,
           Wv, bv, Wm1, bm1, Wm2, bm2):
    del batch
    ages = ages.astype(jnp.float32)
    ei = edge_index.reshape(2, EROWS, SUB)
    zeros_n = jnp.zeros((NPAD, 16), jnp.float32)

    ages_pad = jnp.pad(ages.reshape(N, 1), ((0, NPAD - N), (0, 0)))
    table0 = jnp.concatenate(
        [ages_pad, jnp.ones((NPAD, 1), jnp.float32),
         jnp.zeros((NPAD, 14), jnp.float32)], axis=1)
    eye8 = jnp.eye(8, dtype=jnp.float32)
    sel = lambda j: jnp.kron(
        eye8, jnp.zeros((16, 16), jnp.float32).at[j].set(1.0))
    tile8 = lambda v: jnp.tile(v.reshape(1, 16), (1, 8))

    p0 = _edge_pass0(table0, ei, zeros_n)
    wl0 = Wl0.reshape(HID)
    wr0 = Wr0.reshape(HID)
    table, degf = _dense1(
        p0.reshape(NC, FR, 128), table0.reshape(FR, 128), sel(0), sel(1),
        tile8(wl0[:16]), tile8(wl0[16:]), tile8(wr0[:16]), tile8(wr0[16:]),
        tile8(b0[:16]), tile8(b0[16:]))

    p1 = _edge_pass1(table.reshape(2 * NPAD, 16), ei, zeros_n)

    bd = lambda w: jnp.kron(eye8, w)
    bdA = jnp.stack([bd(Wl1[:16, :16]), bd(Wl1[16:, :16]),
                     bd(Wr1[:16, :16]), bd(Wr1[16:, :16])])
    bdB = jnp.stack([bd(Wl1[:16, 16:]), bd(Wl1[16:, 16:]),
                     bd(Wr1[:16, 16:]), bd(Wr1[16:, 16:])])
    fold = jnp.kron(jnp.ones((8, 1), jnp.float32),
                    jnp.eye(16, dtype=jnp.float32))
    tt = jnp.stack([jnp.asarray(temp), jnp.asarray(t)]).astype(
        jnp.float32).reshape(1, 2)
    res = _dense2(p1.reshape(NC, FR, 128), table, degf,
                  bdA, bdB, tile8(b1[:16]), tile8(b1[16:]), fold,
                  Wv[:16], Wv[16:], bv.reshape(1, 5), tt,
                  Wm1[:5], Wm1[5:], bm1.reshape(1, HID), Wm2,
                  bm2.reshape(1, 1))
    return res.reshape(-1)
